# Initial kernel scaffold; baseline (speedup 1.0000x reference)
import functools

import jax
import jax.numpy as jnp
from jax import lax
from jax.experimental import pallas as pl
from jax.experimental.pallas import tpu as pltpu
from jax.experimental.pallas import tpu_sc as plsc

N = 10000
D = 128
E = 320000
W144 = 144
NC = 2
NS = 16
NWORK = NC * NS
EPW = E // NWORK
CHUNK = 80
NCHUNK = EPW // CHUNK
ROWS_PER_TILE = N // NS

_mesh = plsc.VectorSubcoreMesh(
    core_axis_name="c", subcore_axis_name="s", num_cores=NC, num_subcores=NS)


@functools.partial(
    pl.kernel,
    out_type=jax.ShapeDtypeStruct((NC, N, W144), jnp.float32),
    mesh=_mesh,
    scratch_types=[
        pltpu.VMEM((CHUNK,), jnp.int32),
        pltpu.VMEM((CHUNK,), jnp.int32),
        pltpu.VMEM((CHUNK, D), jnp.float32),
        pltpu.VMEM((CHUNK, D), jnp.float32),
        pltpu.VMEM((CHUNK, W144), jnp.float32),
        pltpu.VMEM((D,), jnp.float32),
        pltpu.VMEM_SHARED((N, W144), jnp.float32),
        pltpu.SemaphoreType.DMA,
    ],
)
def _sc_edge(fs_hbm, fd_hbm, src_hbm, dst_hbm, a_hbm, out_hbm,
             sidx, didx, gs, gd, msg, avec, accum, sem):
    cid = lax.axis_index("c")
    sid = lax.axis_index("s")
    wid = sid * NC + cid

    pltpu.sync_copy(a_hbm, avec)

    def _zrow(i, carry):
        for j in range(W144 // 16):
            msg[i, pl.ds(16 * j, 16)] = jnp.zeros((16,), jnp.float32)
        return carry
    lax.fori_loop(0, CHUNK, _zrow, 0)
    row0 = sid * ROWS_PER_TILE
    for t in range(7):
        pltpu.sync_copy(msg, accum.at[pl.ds(row0 + t * CHUNK, CHUNK)])
    pltpu.sync_copy(msg.at[pl.ds(0, 65)], accum.at[pl.ds(row0 + 560, 65)])
    plsc.subcore_barrier()

    a_sl = tuple(avec[pl.ds(16 * k, 16)] for k in range(D // 16))
    onehot = (lax.iota(jnp.int32, 16) == 0).astype(jnp.float32)

    base0 = wid * EPW

    def _chunk(i, carry):
        base = base0 + i * CHUNK
        pltpu.sync_copy(src_hbm.at[pl.ds(base, CHUNK)], sidx)
        pltpu.sync_copy(dst_hbm.at[pl.ds(base, CHUNK)], didx)
        d1 = pltpu.async_copy(fs_hbm.at[sidx], gs, sem)
        d2 = pltpu.async_copy(fd_hbm.at[didx], gd, sem)
        d1.wait()
        d2.wait()

        def _edge(e, ecarry):
            acc = jnp.zeros((16,), jnp.float32)
            for k in range(D // 16):
                t = gs[e, pl.ds(16 * k, 16)] + gd[e, pl.ds(16 * k, 16)]
                t = jnp.maximum(t, 0.2 * t)
                acc = acc + t * a_sl[k]
            logit = jnp.sum(acc)
            exv = jnp.exp(jnp.full((16,), logit, jnp.float32))
            for k in range(D // 16):
                msg[e, pl.ds(16 * k, 16)] = gs[e, pl.ds(16 * k, 16)] * exv
            msg[e, pl.ds(D, 16)] = exv * onehot
            return ecarry
        lax.fori_loop(0, CHUNK, _edge, 0)

        pltpu.sync_copy(msg, accum.at[didx], add=True)
        return carry
    lax.fori_loop(0, NCHUNK, _chunk, 0)

    plsc.subcore_barrier()
    for t in range(7):
        pltpu.sync_copy(accum.at[pl.ds(row0 + t * CHUNK, CHUNK)],
                        out_hbm.at[cid, pl.ds(row0 + t * CHUNK, CHUNK)])
    pltpu.sync_copy(accum.at[pl.ds(row0 + 560, 65)],
                    out_hbm.at[cid, pl.ds(row0 + 560, 65)])


_RB = 2000


def _tc_matmul(x, w):
    k = w.shape[1]

    def body(x_ref, w_ref, o_ref):
        o_ref[...] = jnp.dot(x_ref[...], w_ref[...],
                             preferred_element_type=jnp.float32)

    return pl.pallas_call(
        body,
        grid=(N // _RB,),
        in_specs=[
            pl.BlockSpec((_RB, x.shape[1]), lambda i: (i, 0)),
            pl.BlockSpec((x.shape[1], k), lambda i: (0, 0)),
        ],
        out_specs=pl.BlockSpec((_RB, k), lambda i: (i, 0)),
        out_shape=jax.ShapeDtypeStruct((N, k), jnp.float32),
    )(x, w)


def _norm(acc_ref, b_ref):
    s = acc_ref[0] + acc_ref[1]
    feat = s[:, :D]
    den = s[:, D:D + 1]
    den = jnp.where(den != 0.0, den, 1.0)
    return feat / den + b_ref[...]


def _tc_norm_mm(acc, b, w):
    def body(acc_ref, b_ref, w_ref, o_ref):
        o_ref[...] = jnp.dot(_norm(acc_ref, b_ref), w_ref[...],
                             preferred_element_type=jnp.float32)

    return pl.pallas_call(
        body,
        grid=(N // _RB,),
        in_specs=[
            pl.BlockSpec((NC, _RB, W144), lambda i: (0, i, 0)),
            pl.BlockSpec((1, D), lambda i: (0, 0)),
            pl.BlockSpec((D, D), lambda i: (0, 0)),
        ],
        out_specs=pl.BlockSpec((_RB, D), lambda i: (i, 0)),
        out_shape=jax.ShapeDtypeStruct((N, D), jnp.float32),
    )(acc, b.reshape(1, D), w)


def _tc_final(acc2b, acc2l, b2b, b2l, wa, wb, outb):
    def body(a1_ref, a2_ref, b1_ref, b2_ref, wa_ref, wb_ref, ob_ref, o_ref):
        x1 = _norm(a1_ref, b1_ref)
        x2 = _norm(a2_ref, b2_ref)
        o_ref[...] = (
            jnp.dot(x1, wa_ref[...], preferred_element_type=jnp.float32)
            + jnp.dot(x2, wb_ref[...], preferred_element_type=jnp.float32)
            + ob_ref[...])

    return pl.pallas_call(
        body,
        grid=(N // _RB,),
        in_specs=[
            pl.BlockSpec((NC, _RB, W144), lambda i: (0, i, 0)),
            pl.BlockSpec((NC, _RB, W144), lambda i: (0, i, 0)),
            pl.BlockSpec((1, D), lambda i: (0, 0)),
            pl.BlockSpec((1, D), lambda i: (0, 0)),
            pl.BlockSpec((D, D), lambda i: (0, 0)),
            pl.BlockSpec((D, D), lambda i: (0, 0)),
            pl.BlockSpec((1, D), lambda i: (0, 0)),
        ],
        out_specs=pl.BlockSpec((_RB, D), lambda i: (i, 0)),
        out_shape=jax.ShapeDtypeStruct((N, D), jnp.float32),
    )(acc2b, acc2l, b2b.reshape(1, D), b2l.reshape(1, D), wa, wb,
      outb.reshape(1, D))


@jax.jit
def kernel(h_user, h_item, rate_src, rate_dst, rb_src, rb_dst, link_src,
           link_dst, W1r_src, W1r_dst, a1r, b1r, W1b_src, W1b_dst, a1b, b1b,
           W2b_src, W2b_dst, a2b, b2b, W2l_src, W2l_dst, a2l, b2l, out_W,
           out_b):
    wu = jnp.concatenate([W1r_src, W1b_dst, W2b_dst, W2l_dst], axis=1)
    wi = jnp.concatenate([W1r_dst, W1b_src], axis=1)
    mu = _tc_matmul(h_user, wu)
    mi = _tc_matmul(h_item, wi)
    fs1r, fd1b, fd2b, fd2l = (mu[:, 0:D], mu[:, D:2 * D],
                              mu[:, 2 * D:3 * D], mu[:, 3 * D:4 * D])
    fd1r, fs1b = mi[:, 0:D], mi[:, D:2 * D]

    acc_r = _sc_edge(fs1r, fd1r, rate_src, rate_dst, a1r)
    acc_b = _sc_edge(fs1b, fd1b, rb_src, rb_dst, a1b)

    fs2b = _tc_norm_mm(acc_r, b1r, W2b_src)
    fs2l = _tc_norm_mm(acc_b, b1b, W2l_src)

    acc_2b = _sc_edge(fs2b, fd2b, rb_src, rb_dst, a2b)
    acc_2l = _sc_edge(fs2l, fd2l, link_src, link_dst, a2l)

    return _tc_final(acc_2b, acc_2l, b2b, b2l,
                     out_W[:D, :], out_W[D:, :], out_b)


# speedup vs baseline: 6.6836x; 6.6836x over previous
import functools

import jax
import jax.numpy as jnp
from jax import lax
from jax.experimental import pallas as pl
from jax.experimental.pallas import tpu as pltpu
from jax.experimental.pallas import tpu_sc as plsc

N = 10000
D = 128
E = 320000
W144 = 144
NC = 2
NS = 16
NWORK = NC * NS
EPW = E // NWORK
CHUNK = 80
NCHUNK = EPW // CHUNK
ROWS_PER_TILE = N // NS

_mesh = plsc.VectorSubcoreMesh(
    core_axis_name="c", subcore_axis_name="s", num_cores=NC, num_subcores=NS)


@functools.partial(
    pl.kernel,
    out_type=jax.ShapeDtypeStruct((NC, N, W144), jnp.float32),
    mesh=_mesh,
    scratch_types=[
        pltpu.VMEM((CHUNK,), jnp.int32),
        pltpu.VMEM((CHUNK,), jnp.int32),
        pltpu.VMEM((CHUNK, D), jnp.float32),
        pltpu.VMEM((CHUNK, D), jnp.float32),
        pltpu.VMEM((CHUNK, W144), jnp.float32),
        pltpu.VMEM((D,), jnp.float32),
        pltpu.VMEM_SHARED((N, W144), jnp.float32),
        pltpu.SemaphoreType.DMA,
    ],
    compiler_params=pltpu.CompilerParams(use_tc_tiling_on_sc=False, needs_layout_passes=False),
)
def _sc_edge(fs_hbm, fd_hbm, src_hbm, dst_hbm, a_hbm, out_hbm,
             sidx, didx, gs, gd, msg, avec, accum, sem):
    cid = lax.axis_index("c")
    sid = lax.axis_index("s")
    wid = sid * NC + cid

    pltpu.sync_copy(a_hbm, avec)

    def _zrow(i, carry):
        for j in range(W144 // 16):
            msg[i, pl.ds(16 * j, 16)] = jnp.zeros((16,), jnp.float32)
        return carry
    lax.fori_loop(0, CHUNK, _zrow, 0)
    row0 = sid * ROWS_PER_TILE
    for t in range(7):
        pltpu.sync_copy(msg, accum.at[pl.ds(row0 + t * CHUNK, CHUNK)])
    pltpu.sync_copy(msg.at[pl.ds(0, 65)], accum.at[pl.ds(row0 + 560, 65)])
    plsc.subcore_barrier()

    a_sl = tuple(avec[pl.ds(16 * k, 16)] for k in range(D // 16))
    onehot = (lax.iota(jnp.int32, 16) == 0).astype(jnp.float32)

    base0 = wid * EPW

    def _chunk(i, carry):
        base = base0 + i * CHUNK
        pltpu.sync_copy(src_hbm.at[pl.ds(base, CHUNK)], sidx)
        pltpu.sync_copy(dst_hbm.at[pl.ds(base, CHUNK)], didx)
        d1 = pltpu.async_copy(fs_hbm.at[sidx], gs, sem)
        d2 = pltpu.async_copy(fd_hbm.at[didx], gd, sem)
        d1.wait()
        d2.wait()

        def _edge(e, ecarry):
            acc = jnp.zeros((16,), jnp.float32)
            for k in range(D // 16):
                t = gs[e, pl.ds(16 * k, 16)] + gd[e, pl.ds(16 * k, 16)]
                t = jnp.maximum(t, 0.2 * t)
                acc = acc + t * a_sl[k]
            logit = jnp.sum(acc)
            exv = jnp.exp(jnp.full((16,), logit, jnp.float32))
            for k in range(D // 16):
                msg[e, pl.ds(16 * k, 16)] = gs[e, pl.ds(16 * k, 16)] * exv
            msg[e, pl.ds(D, 16)] = exv * onehot
            return ecarry
        lax.fori_loop(0, CHUNK, _edge, 0)

        pltpu.sync_copy(msg, accum.at[didx], add=True)
        return carry
    lax.fori_loop(0, NCHUNK, _chunk, 0)

    plsc.subcore_barrier()
    for t in range(7):
        pltpu.sync_copy(accum.at[pl.ds(row0 + t * CHUNK, CHUNK)],
                        out_hbm.at[cid, pl.ds(row0 + t * CHUNK, CHUNK)])
    pltpu.sync_copy(accum.at[pl.ds(row0 + 560, 65)],
                    out_hbm.at[cid, pl.ds(row0 + 560, 65)])


_RB = 2000


def _tc_matmul(x, w):
    k = w.shape[1]

    def body(x_ref, w_ref, o_ref):
        o_ref[...] = jnp.dot(x_ref[...], w_ref[...],
                             preferred_element_type=jnp.float32)

    return pl.pallas_call(
        body,
        grid=(N // _RB,),
        in_specs=[
            pl.BlockSpec((_RB, x.shape[1]), lambda i: (i, 0)),
            pl.BlockSpec((x.shape[1], k), lambda i: (0, 0)),
        ],
        out_specs=pl.BlockSpec((_RB, k), lambda i: (i, 0)),
        out_shape=jax.ShapeDtypeStruct((N, k), jnp.float32),
    )(x, w)


def _norm(acc_ref, b_ref):
    s = acc_ref[0] + acc_ref[1]
    feat = s[:, :D]
    den = s[:, D:D + 1]
    den = jnp.where(den != 0.0, den, 1.0)
    return feat / den + b_ref[...]


def _tc_norm_mm(acc, b, w):
    def body(acc_ref, b_ref, w_ref, o_ref):
        o_ref[...] = jnp.dot(_norm(acc_ref, b_ref), w_ref[...],
                             preferred_element_type=jnp.float32)

    return pl.pallas_call(
        body,
        grid=(N // _RB,),
        in_specs=[
            pl.BlockSpec((NC, _RB, W144), lambda i: (0, i, 0)),
            pl.BlockSpec((1, D), lambda i: (0, 0)),
            pl.BlockSpec((D, D), lambda i: (0, 0)),
        ],
        out_specs=pl.BlockSpec((_RB, D), lambda i: (i, 0)),
        out_shape=jax.ShapeDtypeStruct((N, D), jnp.float32),
    )(acc, b.reshape(1, D), w)


def _tc_final(acc2b, acc2l, b2b, b2l, wa, wb, outb):
    def body(a1_ref, a2_ref, b1_ref, b2_ref, wa_ref, wb_ref, ob_ref, o_ref):
        x1 = _norm(a1_ref, b1_ref)
        x2 = _norm(a2_ref, b2_ref)
        o_ref[...] = (
            jnp.dot(x1, wa_ref[...], preferred_element_type=jnp.float32)
            + jnp.dot(x2, wb_ref[...], preferred_element_type=jnp.float32)
            + ob_ref[...])

    return pl.pallas_call(
        body,
        grid=(N // _RB,),
        in_specs=[
            pl.BlockSpec((NC, _RB, W144), lambda i: (0, i, 0)),
            pl.BlockSpec((NC, _RB, W144), lambda i: (0, i, 0)),
            pl.BlockSpec((1, D), lambda i: (0, 0)),
            pl.BlockSpec((1, D), lambda i: (0, 0)),
            pl.BlockSpec((D, D), lambda i: (0, 0)),
            pl.BlockSpec((D, D), lambda i: (0, 0)),
            pl.BlockSpec((1, D), lambda i: (0, 0)),
        ],
        out_specs=pl.BlockSpec((_RB, D), lambda i: (i, 0)),
        out_shape=jax.ShapeDtypeStruct((N, D), jnp.float32),
    )(acc2b, acc2l, b2b.reshape(1, D), b2l.reshape(1, D), wa, wb,
      outb.reshape(1, D))


@jax.jit
def kernel(h_user, h_item, rate_src, rate_dst, rb_src, rb_dst, link_src,
           link_dst, W1r_src, W1r_dst, a1r, b1r, W1b_src, W1b_dst, a1b, b1b,
           W2b_src, W2b_dst, a2b, b2b, W2l_src, W2l_dst, a2l, b2l, out_W,
           out_b):
    wu = jnp.concatenate([W1r_src, W1b_dst, W2b_dst, W2l_dst], axis=1)
    wi = jnp.concatenate([W1r_dst, W1b_src], axis=1)
    mu = _tc_matmul(h_user, wu)
    mi = _tc_matmul(h_item, wi)
    fs1r, fd1b, fd2b, fd2l = (mu[:, 0:D], mu[:, D:2 * D],
                              mu[:, 2 * D:3 * D], mu[:, 3 * D:4 * D])
    fd1r, fs1b = mi[:, 0:D], mi[:, D:2 * D]

    acc_r = _sc_edge(fs1r, fd1r, rate_src, rate_dst, a1r)
    acc_b = _sc_edge(fs1b, fd1b, rb_src, rb_dst, a1b)

    fs2b = _tc_norm_mm(acc_r, b1r, W2b_src)
    fs2l = _tc_norm_mm(acc_b, b1b, W2l_src)

    acc_2b = _sc_edge(fs2b, fd2b, rb_src, rb_dst, a2b)
    acc_2l = _sc_edge(fs2l, fd2l, link_src, link_dst, a2l)

    return _tc_final(acc_2b, acc_2l, b2b, b2l,
                     out_W[:D, :], out_W[D:, :], out_b)


# speedup vs baseline: 7.9536x; 1.1900x over previous
import functools

import jax
import jax.numpy as jnp
from jax import lax
from jax.experimental import pallas as pl
from jax.experimental.pallas import tpu as pltpu
from jax.experimental.pallas import tpu_sc as plsc

N = 10000
D = 128
E = 320000
W144 = 144
NC = 2
NS = 16
NWORK = NC * NS
EPW = E // NWORK
CHUNK = 40
NCHUNK = EPW // CHUNK
ROWS_PER_TILE = N // NS

_mesh = plsc.VectorSubcoreMesh(
    core_axis_name="c", subcore_axis_name="s", num_cores=NC, num_subcores=NS)


UNROLL = 4


@functools.partial(
    pl.kernel,
    out_type=jax.ShapeDtypeStruct((NC, N, W144), jnp.float32),
    mesh=_mesh,
    scratch_types=[
        pltpu.VMEM((CHUNK,), jnp.int32),
        pltpu.VMEM((CHUNK,), jnp.int32),
        pltpu.VMEM((CHUNK,), jnp.int32),
        pltpu.VMEM((CHUNK,), jnp.int32),
        pltpu.VMEM((CHUNK, D), jnp.float32),
        pltpu.VMEM((CHUNK, D), jnp.float32),
        pltpu.VMEM((CHUNK, D), jnp.float32),
        pltpu.VMEM((CHUNK, D), jnp.float32),
        pltpu.VMEM((CHUNK, W144), jnp.float32),
        pltpu.VMEM((D,), jnp.float32),
        pltpu.VMEM_SHARED((N, W144), jnp.float32),
        pltpu.SemaphoreType.DMA,
        pltpu.SemaphoreType.DMA,
        pltpu.SemaphoreType.DMA,
        pltpu.SemaphoreType.DMA,
    ],
    compiler_params=pltpu.CompilerParams(use_tc_tiling_on_sc=False, needs_layout_passes=False),
)
def _sc_edge(fs_hbm, fd_hbm, src_hbm, dst_hbm, a_hbm, out_hbm,
             sidx0, sidx1, didx0, didx1, gs0, gs1, gd0, gd1, msg0,
             avec, accum, semg0, semg1, semi0, semi1):
    cid = lax.axis_index("c")
    sid = lax.axis_index("s")
    wid = sid * NC + cid

    sidx = (sidx0, sidx1)
    didx = (didx0, didx1)
    gs = (gs0, gs1)
    gd = (gd0, gd1)
    msg = (msg0, msg0)
    semg = (semg0, semg1)
    semi = (semi0, semi1)

    pltpu.sync_copy(a_hbm, avec)

    def _zrow(i, carry):
        for j in range(W144 // 16):
            msg0[i, pl.ds(16 * j, 16)] = jnp.zeros((16,), jnp.float32)
        return carry
    lax.fori_loop(0, CHUNK, _zrow, 0)
    row0 = sid * ROWS_PER_TILE
    for t in range(15):
        pltpu.sync_copy(msg0, accum.at[pl.ds(row0 + t * CHUNK, CHUNK)])
    pltpu.sync_copy(msg0.at[pl.ds(0, 25)], accum.at[pl.ds(row0 + 600, 25)])
    plsc.subcore_barrier()

    a_sl = tuple(avec[pl.ds(16 * k, 16)] for k in range(D // 16))
    onehot = (lax.iota(jnp.int32, 16) == 0).astype(jnp.float32)

    base0 = wid * EPW

    def _idx_issue(i, b):
        base = base0 + i * CHUNK
        pltpu.async_copy(src_hbm.at[pl.ds(base, CHUNK)], sidx[b], semi[b])
        pltpu.async_copy(dst_hbm.at[pl.ds(base, CHUNK)], didx[b], semi[b])

    def _idx_wait(b):
        pltpu.make_async_copy(src_hbm.at[pl.ds(0, CHUNK)], sidx[b], semi[b]).wait()
        pltpu.make_async_copy(dst_hbm.at[pl.ds(0, CHUNK)], didx[b], semi[b]).wait()

    def _gather_issue(b):
        pltpu.async_copy(fs_hbm.at[sidx[b]], gs[b], semg[b])
        pltpu.async_copy(fd_hbm.at[didx[b]], gd[b], semg[b])

    def _gather_wait(b):
        pltpu.make_async_copy(fs_hbm.at[sidx[b]], gs[b], semg[b]).wait()
        pltpu.make_async_copy(fd_hbm.at[didx[b]], gd[b], semg[b]).wait()

    def _compute(b):
        gsb, gdb, msgb = gs[b], gd[b], msg[b]

        def _edge(e, ecarry):
            acc = jnp.zeros((16,), jnp.float32)
            for k in range(D // 16):
                t = gsb[e, pl.ds(16 * k, 16)] + gdb[e, pl.ds(16 * k, 16)]
                t = jnp.maximum(t, 0.2 * t)
                acc = acc + t * a_sl[k]
            logit = jnp.sum(acc)
            exv = jnp.exp(jnp.full((16,), logit, jnp.float32))
            for k in range(D // 16):
                msgb[e, pl.ds(16 * k, 16)] = gsb[e, pl.ds(16 * k, 16)] * exv
            msgb[e, pl.ds(D, 16)] = exv * onehot
            return ecarry
        lax.fori_loop(0, CHUNK, _edge, 0)

    def _pipe_chunk(i, b, gather_next=True, prefetch=True):
        nb = 1 - b
        if gather_next:
            _idx_wait(nb)
            _gather_issue(nb)
        _gather_wait(b)
        _compute(b)
        pltpu.sync_copy(msg[b], accum.at[didx[b]], add=True)
        if prefetch:
            _idx_issue(i + 2, b)

    pltpu.sync_copy(src_hbm.at[pl.ds(base0, CHUNK)], sidx0)
    pltpu.sync_copy(dst_hbm.at[pl.ds(base0, CHUNK)], didx0)
    _gather_issue(0)
    _idx_issue(1, 1)

    def _pair(k, carry):
        _pipe_chunk(2 * k, 0)
        _pipe_chunk(2 * k + 1, 1)
        return carry
    lax.fori_loop(0, (NCHUNK - 4) // 2, _pair, 0)

    _pipe_chunk(NCHUNK - 4, 0)
    _pipe_chunk(NCHUNK - 3, 1)
    _pipe_chunk(NCHUNK - 2, 0, prefetch=False)
    _pipe_chunk(NCHUNK - 1, 1, gather_next=False, prefetch=False)

    plsc.subcore_barrier()
    for t in range(15):
        pltpu.sync_copy(accum.at[pl.ds(row0 + t * CHUNK, CHUNK)],
                        out_hbm.at[cid, pl.ds(row0 + t * CHUNK, CHUNK)])
    pltpu.sync_copy(accum.at[pl.ds(row0 + 600, 25)],
                    out_hbm.at[cid, pl.ds(row0 + 600, 25)])


_RB = 2000


def _tc_matmul(x, w):
    k = w.shape[1]

    def body(x_ref, w_ref, o_ref):
        o_ref[...] = jnp.dot(x_ref[...], w_ref[...],
                             preferred_element_type=jnp.float32)

    return pl.pallas_call(
        body,
        grid=(N // _RB,),
        in_specs=[
            pl.BlockSpec((_RB, x.shape[1]), lambda i: (i, 0)),
            pl.BlockSpec((x.shape[1], k), lambda i: (0, 0)),
        ],
        out_specs=pl.BlockSpec((_RB, k), lambda i: (i, 0)),
        out_shape=jax.ShapeDtypeStruct((N, k), jnp.float32),
    )(x, w)


def _norm(acc_ref, b_ref):
    s = acc_ref[0] + acc_ref[1]
    feat = s[:, :D]
    den = s[:, D:D + 1]
    den = jnp.where(den != 0.0, den, 1.0)
    return feat / den + b_ref[...]


def _tc_norm_mm(acc, b, w):
    def body(acc_ref, b_ref, w_ref, o_ref):
        o_ref[...] = jnp.dot(_norm(acc_ref, b_ref), w_ref[...],
                             preferred_element_type=jnp.float32)

    return pl.pallas_call(
        body,
        grid=(N // _RB,),
        in_specs=[
            pl.BlockSpec((NC, _RB, W144), lambda i: (0, i, 0)),
            pl.BlockSpec((1, D), lambda i: (0, 0)),
            pl.BlockSpec((D, D), lambda i: (0, 0)),
        ],
        out_specs=pl.BlockSpec((_RB, D), lambda i: (i, 0)),
        out_shape=jax.ShapeDtypeStruct((N, D), jnp.float32),
    )(acc, b.reshape(1, D), w)


def _tc_final(acc2b, acc2l, b2b, b2l, wa, wb, outb):
    def body(a1_ref, a2_ref, b1_ref, b2_ref, wa_ref, wb_ref, ob_ref, o_ref):
        x1 = _norm(a1_ref, b1_ref)
        x2 = _norm(a2_ref, b2_ref)
        o_ref[...] = (
            jnp.dot(x1, wa_ref[...], preferred_element_type=jnp.float32)
            + jnp.dot(x2, wb_ref[...], preferred_element_type=jnp.float32)
            + ob_ref[...])

    return pl.pallas_call(
        body,
        grid=(N // _RB,),
        in_specs=[
            pl.BlockSpec((NC, _RB, W144), lambda i: (0, i, 0)),
            pl.BlockSpec((NC, _RB, W144), lambda i: (0, i, 0)),
            pl.BlockSpec((1, D), lambda i: (0, 0)),
            pl.BlockSpec((1, D), lambda i: (0, 0)),
            pl.BlockSpec((D, D), lambda i: (0, 0)),
            pl.BlockSpec((D, D), lambda i: (0, 0)),
            pl.BlockSpec((1, D), lambda i: (0, 0)),
        ],
        out_specs=pl.BlockSpec((_RB, D), lambda i: (i, 0)),
        out_shape=jax.ShapeDtypeStruct((N, D), jnp.float32),
    )(acc2b, acc2l, b2b.reshape(1, D), b2l.reshape(1, D), wa, wb,
      outb.reshape(1, D))


@jax.jit
def kernel(h_user, h_item, rate_src, rate_dst, rb_src, rb_dst, link_src,
           link_dst, W1r_src, W1r_dst, a1r, b1r, W1b_src, W1b_dst, a1b, b1b,
           W2b_src, W2b_dst, a2b, b2b, W2l_src, W2l_dst, a2l, b2l, out_W,
           out_b):
    wu = jnp.concatenate([W1r_src, W1b_dst, W2b_dst, W2l_dst], axis=1)
    wi = jnp.concatenate([W1r_dst, W1b_src], axis=1)
    mu = _tc_matmul(h_user, wu)
    mi = _tc_matmul(h_item, wi)
    fs1r, fd1b, fd2b, fd2l = (mu[:, 0:D], mu[:, D:2 * D],
                              mu[:, 2 * D:3 * D], mu[:, 3 * D:4 * D])
    fd1r, fs1b = mi[:, 0:D], mi[:, D:2 * D]

    acc_r = _sc_edge(fs1r, fd1r, rate_src, rate_dst, a1r)
    acc_b = _sc_edge(fs1b, fd1b, rb_src, rb_dst, a1b)

    fs2b = _tc_norm_mm(acc_r, b1r, W2b_src)
    fs2l = _tc_norm_mm(acc_b, b1b, W2l_src)

    acc_2b = _sc_edge(fs2b, fd2b, rb_src, rb_dst, a2b)
    acc_2l = _sc_edge(fs2l, fd2l, link_src, link_dst, a2l)

    return _tc_final(acc_2b, acc_2l, b2b, b2l,
                     out_W[:D, :], out_W[D:, :], out_b)


# speedup vs baseline: 17.5723x; 2.2094x over previous
import functools

import jax
import jax.numpy as jnp
from jax import lax
from jax.experimental import pallas as pl
from jax.experimental.pallas import tpu as pltpu
from jax.experimental.pallas import tpu_sc as plsc

N = 10000
D = 128
E = 320000
W144 = 144
NC = 2
NS = 16
NWORK = NC * NS
EPW = E // NWORK
CHUNK = 40
NCHUNK = EPW // CHUNK
ROWS_PER_TILE = N // NS

_mesh = plsc.VectorSubcoreMesh(
    core_axis_name="c", subcore_axis_name="s", num_cores=NC, num_subcores=NS)


UNROLL = 4


@functools.partial(
    pl.kernel,
    out_type=jax.ShapeDtypeStruct((NC, N, W144), jnp.float32),
    mesh=_mesh,
    scratch_types=[
        pltpu.VMEM((CHUNK,), jnp.int32),
        pltpu.VMEM((CHUNK,), jnp.int32),
        pltpu.VMEM((CHUNK,), jnp.int32),
        pltpu.VMEM((CHUNK,), jnp.int32),
        pltpu.VMEM((CHUNK, D), jnp.float32),
        pltpu.VMEM((CHUNK, D), jnp.float32),
        pltpu.VMEM((CHUNK, D), jnp.float32),
        pltpu.VMEM((CHUNK, D), jnp.float32),
        pltpu.VMEM((CHUNK, W144), jnp.float32),
        pltpu.VMEM((D,), jnp.float32),
        pltpu.VMEM_SHARED((N, W144), jnp.float32),
        pltpu.SemaphoreType.DMA,
        pltpu.SemaphoreType.DMA,
        pltpu.SemaphoreType.DMA,
        pltpu.SemaphoreType.DMA,
    ],
    compiler_params=pltpu.CompilerParams(use_tc_tiling_on_sc=False, needs_layout_passes=False),
)
def _sc_edge(fs_hbm, fd_hbm, src_hbm, dst_hbm, a_hbm, out_hbm,
             sidx0, sidx1, didx0, didx1, gs0, gs1, gd0, gd1, msg0,
             avec, accum, semg0, semg1, semi0, semi1):
    cid = lax.axis_index("c")
    sid = lax.axis_index("s")
    wid = sid * NC + cid

    sidx = (sidx0, sidx1)
    didx = (didx0, didx1)
    gs = (gs0, gs1)
    gd = (gd0, gd1)
    msg = (msg0, msg0)
    semg = (semg0, semg1)
    semi = (semi0, semi1)

    pltpu.sync_copy(a_hbm, avec)

    def _zrow(i, carry):
        for j in range(W144 // 16):
            msg0[i, pl.ds(16 * j, 16)] = jnp.zeros((16,), jnp.float32)
        return carry
    lax.fori_loop(0, CHUNK, _zrow, 0)
    row0 = sid * ROWS_PER_TILE
    for t in range(15):
        pltpu.sync_copy(msg0, accum.at[pl.ds(row0 + t * CHUNK, CHUNK)])
    pltpu.sync_copy(msg0.at[pl.ds(0, 25)], accum.at[pl.ds(row0 + 600, 25)])
    plsc.subcore_barrier()

    a_sl = tuple(avec[pl.ds(16 * k, 16)] for k in range(D // 16))
    onehot = (lax.iota(jnp.int32, 16) == 0).astype(jnp.float32)

    base0 = wid * EPW

    def _idx_issue(i, b):
        base = base0 + i * CHUNK
        pltpu.async_copy(src_hbm.at[pl.ds(base, CHUNK)], sidx[b], semi[b])
        pltpu.async_copy(dst_hbm.at[pl.ds(base, CHUNK)], didx[b], semi[b])

    def _idx_wait(b):
        pltpu.make_async_copy(src_hbm.at[pl.ds(0, CHUNK)], sidx[b], semi[b]).wait()
        pltpu.make_async_copy(dst_hbm.at[pl.ds(0, CHUNK)], didx[b], semi[b]).wait()

    def _gather_issue(b):
        pltpu.async_copy(fs_hbm.at[sidx[b]], gs[b], semg[b])
        pltpu.async_copy(fd_hbm.at[didx[b]], gd[b], semg[b])

    def _gather_wait(b):
        pltpu.make_async_copy(fs_hbm.at[sidx[b]], gs[b], semg[b]).wait()
        pltpu.make_async_copy(fd_hbm.at[didx[b]], gd[b], semg[b]).wait()

    def _compute(b):
        gsb, gdb, msgb = gs[b], gd[b], msg[b]

        def _edge(eg, ecarry):
            es = [eg * UNROLL + u for u in range(UNROLL)]
            rows = []
            for e in es:
                row = [gsb[e, pl.ds(16 * k, 16)] for k in range(D // 16)]
                acc0 = jnp.zeros((16,), jnp.float32)
                acc1 = jnp.zeros((16,), jnp.float32)
                for k in range(D // 16):
                    t = row[k] + gdb[e, pl.ds(16 * k, 16)]
                    t = jnp.maximum(t, 0.2 * t)
                    if k % 2 == 0:
                        acc0 = acc0 + t * a_sl[k]
                    else:
                        acc1 = acc1 + t * a_sl[k]
                rows.append((row, acc0 + acc1))
            for e, (row, acc) in zip(es, rows):
                logit = jnp.sum(acc)
                exv = jnp.exp(jnp.full((16,), logit, jnp.float32))
                for k in range(D // 16):
                    msgb[e, pl.ds(16 * k, 16)] = row[k] * exv
                msgb[e, pl.ds(D, 16)] = exv * onehot
            return ecarry
        lax.fori_loop(0, CHUNK // UNROLL, _edge, 0)

    def _pipe_chunk(i, b, gather_next=True, prefetch=True):
        nb = 1 - b
        if gather_next:
            _idx_wait(nb)
            _gather_issue(nb)
        _gather_wait(b)
        _compute(b)
        pltpu.sync_copy(msg[b], accum.at[didx[b]], add=True)
        if prefetch:
            _idx_issue(i + 2, b)

    pltpu.sync_copy(src_hbm.at[pl.ds(base0, CHUNK)], sidx0)
    pltpu.sync_copy(dst_hbm.at[pl.ds(base0, CHUNK)], didx0)
    _gather_issue(0)
    _idx_issue(1, 1)

    def _pair(k, carry):
        _pipe_chunk(2 * k, 0)
        _pipe_chunk(2 * k + 1, 1)
        return carry
    lax.fori_loop(0, (NCHUNK - 4) // 2, _pair, 0)

    _pipe_chunk(NCHUNK - 4, 0)
    _pipe_chunk(NCHUNK - 3, 1)
    _pipe_chunk(NCHUNK - 2, 0, prefetch=False)
    _pipe_chunk(NCHUNK - 1, 1, gather_next=False, prefetch=False)

    plsc.subcore_barrier()
    for t in range(15):
        pltpu.sync_copy(accum.at[pl.ds(row0 + t * CHUNK, CHUNK)],
                        out_hbm.at[cid, pl.ds(row0 + t * CHUNK, CHUNK)])
    pltpu.sync_copy(accum.at[pl.ds(row0 + 600, 25)],
                    out_hbm.at[cid, pl.ds(row0 + 600, 25)])


_RB = 2000


def _tc_matmul(x, w):
    k = w.shape[1]

    def body(x_ref, w_ref, o_ref):
        o_ref[...] = jnp.dot(x_ref[...], w_ref[...],
                             preferred_element_type=jnp.float32)

    return pl.pallas_call(
        body,
        grid=(N // _RB,),
        in_specs=[
            pl.BlockSpec((_RB, x.shape[1]), lambda i: (i, 0)),
            pl.BlockSpec((x.shape[1], k), lambda i: (0, 0)),
        ],
        out_specs=pl.BlockSpec((_RB, k), lambda i: (i, 0)),
        out_shape=jax.ShapeDtypeStruct((N, k), jnp.float32),
    )(x, w)


def _norm(acc_ref, b_ref):
    s = acc_ref[0] + acc_ref[1]
    feat = s[:, :D]
    den = s[:, D:D + 1]
    den = jnp.where(den != 0.0, den, 1.0)
    return feat / den + b_ref[...]


def _tc_norm_mm(acc, b, w):
    def body(acc_ref, b_ref, w_ref, o_ref):
        o_ref[...] = jnp.dot(_norm(acc_ref, b_ref), w_ref[...],
                             preferred_element_type=jnp.float32)

    return pl.pallas_call(
        body,
        grid=(N // _RB,),
        in_specs=[
            pl.BlockSpec((NC, _RB, W144), lambda i: (0, i, 0)),
            pl.BlockSpec((1, D), lambda i: (0, 0)),
            pl.BlockSpec((D, D), lambda i: (0, 0)),
        ],
        out_specs=pl.BlockSpec((_RB, D), lambda i: (i, 0)),
        out_shape=jax.ShapeDtypeStruct((N, D), jnp.float32),
    )(acc, b.reshape(1, D), w)


def _tc_final(acc2b, acc2l, b2b, b2l, wa, wb, outb):
    def body(a1_ref, a2_ref, b1_ref, b2_ref, wa_ref, wb_ref, ob_ref, o_ref):
        x1 = _norm(a1_ref, b1_ref)
        x2 = _norm(a2_ref, b2_ref)
        o_ref[...] = (
            jnp.dot(x1, wa_ref[...], preferred_element_type=jnp.float32)
            + jnp.dot(x2, wb_ref[...], preferred_element_type=jnp.float32)
            + ob_ref[...])

    return pl.pallas_call(
        body,
        grid=(N // _RB,),
        in_specs=[
            pl.BlockSpec((NC, _RB, W144), lambda i: (0, i, 0)),
            pl.BlockSpec((NC, _RB, W144), lambda i: (0, i, 0)),
            pl.BlockSpec((1, D), lambda i: (0, 0)),
            pl.BlockSpec((1, D), lambda i: (0, 0)),
            pl.BlockSpec((D, D), lambda i: (0, 0)),
            pl.BlockSpec((D, D), lambda i: (0, 0)),
            pl.BlockSpec((1, D), lambda i: (0, 0)),
        ],
        out_specs=pl.BlockSpec((_RB, D), lambda i: (i, 0)),
        out_shape=jax.ShapeDtypeStruct((N, D), jnp.float32),
    )(acc2b, acc2l, b2b.reshape(1, D), b2l.reshape(1, D), wa, wb,
      outb.reshape(1, D))


@jax.jit
def kernel(h_user, h_item, rate_src, rate_dst, rb_src, rb_dst, link_src,
           link_dst, W1r_src, W1r_dst, a1r, b1r, W1b_src, W1b_dst, a1b, b1b,
           W2b_src, W2b_dst, a2b, b2b, W2l_src, W2l_dst, a2l, b2l, out_W,
           out_b):
    wu = jnp.concatenate([W1r_src, W1b_dst, W2b_dst, W2l_dst], axis=1)
    wi = jnp.concatenate([W1r_dst, W1b_src], axis=1)
    mu = _tc_matmul(h_user, wu)
    mi = _tc_matmul(h_item, wi)
    fs1r, fd1b, fd2b, fd2l = (mu[:, 0:D], mu[:, D:2 * D],
                              mu[:, 2 * D:3 * D], mu[:, 3 * D:4 * D])
    fd1r, fs1b = mi[:, 0:D], mi[:, D:2 * D]

    acc_r = _sc_edge(fs1r, fd1r, rate_src, rate_dst, a1r)
    acc_b = _sc_edge(fs1b, fd1b, rb_src, rb_dst, a1b)

    fs2b = _tc_norm_mm(acc_r, b1r, W2b_src)
    fs2l = _tc_norm_mm(acc_b, b1b, W2l_src)

    acc_2b = _sc_edge(fs2b, fd2b, rb_src, rb_dst, a2b)
    acc_2l = _sc_edge(fs2l, fd2l, link_src, link_dst, a2l)

    return _tc_final(acc_2b, acc_2l, b2b, b2l,
                     out_W[:D, :], out_W[D:, :], out_b)


# speedup vs baseline: 19.7559x; 1.1243x over previous
import functools

import jax
import jax.numpy as jnp
import numpy as np
from jax import lax
from jax.experimental import pallas as pl
from jax.experimental.pallas import tpu as pltpu
from jax.experimental.pallas import tpu_sc as plsc

N = 10000
D = 128
E = 320000
W144 = 144
NC = 2
NS = 16
NWORK = NC * NS
EPW = E // NWORK
CHUNK = 40
NCHUNK = EPW // CHUNK
ROWS_PER_TILE = N // NS

_mesh = plsc.VectorSubcoreMesh(
    core_axis_name="c", subcore_axis_name="s", num_cores=NC, num_subcores=NS)


UNROLL = 4


@functools.partial(
    pl.kernel,
    out_type=jax.ShapeDtypeStruct((NC, N, W144), jnp.float32),
    mesh=_mesh,
    scratch_types=(
        [pltpu.VMEM((CHUNK,), jnp.int32)] * 4
        + [pltpu.VMEM((CHUNK,), jnp.int32)] * 4
        + [pltpu.VMEM((CHUNK, D), jnp.bfloat16)] * 2
        + [pltpu.VMEM((CHUNK, D), jnp.bfloat16)] * 2
        + [pltpu.VMEM((CHUNK, W144), jnp.float32)] * 2
        + [pltpu.VMEM((D,), jnp.float32)]
        + [pltpu.VMEM_SHARED((N, W144), jnp.float32)]
        + [pltpu.SemaphoreType.DMA] * 6
    ),
    compiler_params=pltpu.CompilerParams(use_tc_tiling_on_sc=False, needs_layout_passes=False),
)
def _sc_edge(fs_hbm, fd_hbm, src_hbm, dst_hbm, a_hbm, out_hbm,
             sidx0, sidx1, sidx2, sidx3, didx0, didx1, didx2, didx3,
             gs0, gs1, gd0, gd1, msg0, msg1,
             avec, accum, semg0, semg1, semi0, semi1, sems0, sems1):
    cid = lax.axis_index("c")
    sid = lax.axis_index("s")
    wid = sid * NC + cid

    sidx = (sidx0, sidx1, sidx2, sidx3)
    didx = (didx0, didx1, didx2, didx3)
    gs = (gs0, gs1)
    gd = (gd0, gd1)
    msg = (msg0, msg1)
    semg = (semg0, semg1)
    semi = (semi0, semi1)
    sems = (sems0, sems1)

    pltpu.sync_copy(a_hbm, avec)

    def _zrow(i, carry):
        for j in range(W144 // 16):
            msg0[i, pl.ds(16 * j, 16)] = jnp.zeros((16,), jnp.float32)
        return carry
    lax.fori_loop(0, CHUNK, _zrow, 0)
    row0 = sid * ROWS_PER_TILE
    for t in range(15):
        pltpu.sync_copy(msg0, accum.at[pl.ds(row0 + t * CHUNK, CHUNK)])
    pltpu.sync_copy(msg0.at[pl.ds(0, 25)], accum.at[pl.ds(row0 + 600, 25)])
    plsc.subcore_barrier()

    a_sl = tuple(avec[pl.ds(16 * k, 16)] for k in range(D // 16))

    base0 = wid * EPW

    def _idx_issue(i, b4, b2):
        base = base0 + i * CHUNK
        pltpu.async_copy(src_hbm.at[pl.ds(base, CHUNK)], sidx[b4], semi[b2])
        pltpu.async_copy(dst_hbm.at[pl.ds(base, CHUNK)], didx[b4], semi[b2])

    def _idx_wait(b4, b2):
        pltpu.make_async_copy(src_hbm.at[pl.ds(0, CHUNK)], sidx[b4], semi[b2]).wait()
        pltpu.make_async_copy(dst_hbm.at[pl.ds(0, CHUNK)], didx[b4], semi[b2]).wait()

    def _gather_issue(b4, b2):
        pltpu.async_copy(fs_hbm.at[sidx[b4]], gs[b2], semg[b2])
        pltpu.async_copy(fd_hbm.at[didx[b4]], gd[b2], semg[b2])

    def _gather_wait(b4, b2):
        pltpu.make_async_copy(fs_hbm.at[sidx[b4]], gs[b2], semg[b2]).wait()
        pltpu.make_async_copy(fd_hbm.at[didx[b4]], gd[b2], semg[b2]).wait()

    def _scatter_issue(b2, b4):
        pltpu.async_copy(msg[b2], accum.at[didx[b4]], sems[b2], add=True)

    def _scatter_wait(b2, b4):
        pltpu.make_async_copy(msg[b2], accum.at[didx[b4]], sems[b2]).wait()

    _HI = jnp.int32(-65536)

    def _unpack(v32):
        vi = plsc.bitcast(v32, jnp.int32)
        ev = plsc.bitcast(lax.shift_left(vi, 16), jnp.float32)
        od = plsc.bitcast(vi & _HI, jnp.float32)
        return ev, od

    def _compute(b):
        gsb, gdb, msgb = gs[b], gd[b], msg[b]

        def _edge(eg, ecarry):
            es = [eg * UNROLL + u for u in range(UNROLL)]
            infos = []
            for e in es:
                acc0 = jnp.zeros((16,), jnp.float32)
                acc1 = jnp.zeros((16,), jnp.float32)
                halves = []
                for g in range(D // 32):
                    se, so = _unpack(gsb[e, pl.ds(32 * g, 32)])
                    de, do = _unpack(gdb[e, pl.ds(32 * g, 32)])
                    te = se + de
                    te = jnp.maximum(te, 0.2 * te)
                    acc0 = acc0 + te * a_sl[2 * g]
                    to = so + do
                    to = jnp.maximum(to, 0.2 * to)
                    acc1 = acc1 + to * a_sl[2 * g + 1]
                    halves.append((se, so))
                infos.append((halves, acc0 + acc1))
            for e, (halves, acc) in zip(es, infos):
                logit = jnp.sum(acc)
                exv = jnp.exp(jnp.full((16,), logit, jnp.float32))
                for g in range(D // 32):
                    msgb[e, pl.ds(32 * g, 16)] = halves[g][0] * exv
                    msgb[e, pl.ds(32 * g + 16, 16)] = halves[g][1] * exv
                msgb[e, pl.ds(D, 16)] = exv
            return ecarry
        lax.fori_loop(0, CHUNK // UNROLL, _edge, 0)

    def _chunk_step(i, b2, b4, has_next=True, has_next2=True,
                    wait_prev_scatter=True):
        if has_next:
            _idx_wait((b4 + 1) % 4, 1 - b2)
            _gather_issue((b4 + 1) % 4, 1 - b2)
        _gather_wait(b4, b2)
        if wait_prev_scatter:
            _scatter_wait(b2, (b4 + 2) % 4)
        _compute(b2)
        _scatter_issue(b2, b4)
        if has_next2:
            _idx_issue(i + 2, (b4 + 2) % 4, b2)

    pltpu.sync_copy(src_hbm.at[pl.ds(base0, CHUNK)], sidx0)
    pltpu.sync_copy(dst_hbm.at[pl.ds(base0, CHUNK)], didx0)
    _gather_issue(0, 0)
    _idx_issue(1, 1, 1)
    _chunk_step(0, 0, 0, wait_prev_scatter=False)
    _chunk_step(1, 1, 1, wait_prev_scatter=False)

    def _quad(k, carry):
        i = 2 + 4 * k
        _chunk_step(i, 0, 2)
        _chunk_step(i + 1, 1, 3)
        _chunk_step(i + 2, 0, 0)
        _chunk_step(i + 3, 1, 1)
        return carry
    lax.fori_loop(0, (NCHUNK - 6) // 4, _quad, 0)

    _chunk_step(NCHUNK - 4, 0, 2)
    _chunk_step(NCHUNK - 3, 1, 3)
    _chunk_step(NCHUNK - 2, 0, 0, has_next2=False)
    _chunk_step(NCHUNK - 1, 1, 1, has_next=False, has_next2=False)
    _scatter_wait(0, 0)
    _scatter_wait(1, 1)

    plsc.subcore_barrier()
    for t in range(15):
        pltpu.sync_copy(accum.at[pl.ds(row0 + t * CHUNK, CHUNK)],
                        out_hbm.at[cid, pl.ds(row0 + t * CHUNK, CHUNK)])
    pltpu.sync_copy(accum.at[pl.ds(row0 + 600, 25)],
                    out_hbm.at[cid, pl.ds(row0 + 600, 25)])


_RB = 2000


def _tc_matmul(x, w):
    k = w.shape[1]

    def body(x_ref, w_ref, o_ref):
        o_ref[...] = jnp.dot(x_ref[...], w_ref[...],
                             preferred_element_type=jnp.float32)

    return pl.pallas_call(
        body,
        grid=(N // _RB,),
        in_specs=[
            pl.BlockSpec((_RB, x.shape[1]), lambda i: (i, 0)),
            pl.BlockSpec((x.shape[1], k), lambda i: (0, 0)),
        ],
        out_specs=pl.BlockSpec((_RB, k), lambda i: (i, 0)),
        out_shape=jax.ShapeDtypeStruct((N, k), jnp.float32),
    )(x, w)


def _norm(acc_ref, b_ref):
    s = acc_ref[0] + acc_ref[1]
    feat = s[:, :D]
    den = s[:, D:D + 1]
    den = jnp.where(den != 0.0, den, 1.0)
    return feat / den + b_ref[...]


def _tc_norm_mm(acc, b, w):
    def body(acc_ref, b_ref, w_ref, o_ref):
        o_ref[...] = jnp.dot(_norm(acc_ref, b_ref), w_ref[...],
                             preferred_element_type=jnp.float32)

    return pl.pallas_call(
        body,
        grid=(N // _RB,),
        in_specs=[
            pl.BlockSpec((NC, _RB, W144), lambda i: (0, i, 0)),
            pl.BlockSpec((1, D), lambda i: (0, 0)),
            pl.BlockSpec((D, D), lambda i: (0, 0)),
        ],
        out_specs=pl.BlockSpec((_RB, D), lambda i: (i, 0)),
        out_shape=jax.ShapeDtypeStruct((N, D), jnp.float32),
    )(acc, b.reshape(1, D), w)


def _tc_final(acc2b, acc2l, b2b, b2l, wa, wb, outb):
    def body(a1_ref, a2_ref, b1_ref, b2_ref, wa_ref, wb_ref, ob_ref, o_ref):
        x1 = _norm(a1_ref, b1_ref)
        x2 = _norm(a2_ref, b2_ref)
        o_ref[...] = (
            jnp.dot(x1, wa_ref[...], preferred_element_type=jnp.float32)
            + jnp.dot(x2, wb_ref[...], preferred_element_type=jnp.float32)
            + ob_ref[...])

    return pl.pallas_call(
        body,
        grid=(N // _RB,),
        in_specs=[
            pl.BlockSpec((NC, _RB, W144), lambda i: (0, i, 0)),
            pl.BlockSpec((NC, _RB, W144), lambda i: (0, i, 0)),
            pl.BlockSpec((1, D), lambda i: (0, 0)),
            pl.BlockSpec((1, D), lambda i: (0, 0)),
            pl.BlockSpec((D, D), lambda i: (0, 0)),
            pl.BlockSpec((D, D), lambda i: (0, 0)),
            pl.BlockSpec((1, D), lambda i: (0, 0)),
        ],
        out_specs=pl.BlockSpec((_RB, D), lambda i: (i, 0)),
        out_shape=jax.ShapeDtypeStruct((N, D), jnp.float32),
    )(acc2b, acc2l, b2b.reshape(1, D), b2l.reshape(1, D), wa, wb,
      outb.reshape(1, D))


_PERM = np.concatenate(
    [np.concatenate([np.arange(32 * g, 32 * g + 32, 2),
                     np.arange(32 * g + 1, 32 * g + 32, 2)])
     for g in range(D // 32)])


@jax.jit
def kernel(h_user, h_item, rate_src, rate_dst, rb_src, rb_dst, link_src,
           link_dst, W1r_src, W1r_dst, a1r, b1r, W1b_src, W1b_dst, a1b, b1b,
           W2b_src, W2b_dst, a2b, b2b, W2l_src, W2l_dst, a2l, b2l, out_W,
           out_b):
    wu = jnp.concatenate([W1r_src, W1b_dst, W2b_dst, W2l_dst], axis=1)
    wi = jnp.concatenate([W1r_dst, W1b_src], axis=1)
    mu = _tc_matmul(h_user, wu).astype(jnp.bfloat16)
    mi = _tc_matmul(h_item, wi).astype(jnp.bfloat16)
    fs1r, fd1b, fd2b, fd2l = (mu[:, 0:D], mu[:, D:2 * D],
                              mu[:, 2 * D:3 * D], mu[:, 3 * D:4 * D])
    fd1r, fs1b = mi[:, 0:D], mi[:, D:2 * D]

    acc_r = _sc_edge(fs1r, fd1r, rate_src, rate_dst, a1r[_PERM])
    acc_b = _sc_edge(fs1b, fd1b, rb_src, rb_dst, a1b[_PERM])

    fs2b = _tc_norm_mm(acc_r, b1r[_PERM], W2b_src[_PERM, :]).astype(jnp.bfloat16)
    fs2l = _tc_norm_mm(acc_b, b1b[_PERM], W2l_src[_PERM, :]).astype(jnp.bfloat16)

    acc_2b = _sc_edge(fs2b, fd2b, rb_src, rb_dst, a2b[_PERM])
    acc_2l = _sc_edge(fs2l, fd2l, link_src, link_dst, a2l[_PERM])

    return _tc_final(acc_2b, acc_2l, b2b[_PERM], b2l[_PERM],
                     out_W[:D, :][_PERM, :], out_W[D:, :][_PERM, :], out_b)


# speedup vs baseline: 19.8768x; 1.0061x over previous
import functools

import jax
import jax.numpy as jnp
from jax import lax
from jax.experimental import pallas as pl
from jax.experimental.pallas import tpu as pltpu
from jax.experimental.pallas import tpu_sc as plsc

N = 10000
D = 128
E = 320000
W144 = 144
NC = 2
NS = 16
NWORK = NC * NS
EPW = E // NWORK
CHUNK = 40
NCHUNK = EPW // CHUNK
ROWS_PER_TILE = N // NS

_mesh = plsc.VectorSubcoreMesh(
    core_axis_name="c", subcore_axis_name="s", num_cores=NC, num_subcores=NS)


UNROLL = 4


@functools.partial(
    pl.kernel,
    out_type=jax.ShapeDtypeStruct((NC, N, W144), jnp.float32),
    mesh=_mesh,
    scratch_types=(
        [pltpu.VMEM((2 * CHUNK,), jnp.int32)] * 4
        + [pltpu.VMEM((CHUNK,), jnp.int32)] * 2
        + [pltpu.VMEM((2 * CHUNK, D), jnp.float32)] * 2
        + [pltpu.VMEM((CHUNK, W144), jnp.float32)] * 2
        + [pltpu.VMEM((D,), jnp.float32)]
        + [pltpu.VMEM_SHARED((N, W144), jnp.float32)]
        + [pltpu.SemaphoreType.DMA] * 6
    ),
    compiler_params=pltpu.CompilerParams(use_tc_tiling_on_sc=False, needs_layout_passes=False),
)
def _sc_edge(f2_hbm, ci_hbm, a_hbm, out_hbm,
             cidx0, cidx1, cidx2, cidx3, didx0, didx1, gb0, gb1, msg0, msg1,
             avec, accum, semg0, semg1, semi0, semi1, sems0, sems1):
    cid = lax.axis_index("c")
    sid = lax.axis_index("s")
    wid = sid * NC + cid

    cidx = (cidx0, cidx1, cidx2, cidx3)
    didx = (didx0, didx1)
    gb = (gb0, gb1)
    msg = (msg0, msg1)
    semg = (semg0, semg1)
    semi = (semi0, semi1)
    sems = (sems0, sems1)

    pltpu.sync_copy(a_hbm, avec)

    def _zrow(i, carry):
        for j in range(W144 // 16):
            msg0[i, pl.ds(16 * j, 16)] = jnp.zeros((16,), jnp.float32)
        return carry
    lax.fori_loop(0, CHUNK, _zrow, 0)
    row0 = sid * ROWS_PER_TILE
    for t in range(15):
        pltpu.sync_copy(msg0, accum.at[pl.ds(row0 + t * CHUNK, CHUNK)])
    pltpu.sync_copy(msg0.at[pl.ds(0, 25)], accum.at[pl.ds(row0 + 600, 25)])
    plsc.subcore_barrier()

    a_sl = tuple(avec[pl.ds(16 * k, 16)] for k in range(D // 16))

    base0 = wid * EPW

    ord0 = wid * NCHUNK

    def _idx_issue(i, b4, b2):
        pltpu.async_copy(ci_hbm.at[ord0 + i], cidx[b4], semi[b2])

    def _idx_wait(b4, b2):
        pltpu.make_async_copy(ci_hbm.at[0], cidx[b4], semi[b2]).wait()

    def _gather_issue(b4, b2):
        pltpu.async_copy(f2_hbm.at[cidx[b4]], gb[b2], semg[b2])

    def _gather_wait(b4, b2):
        pltpu.make_async_copy(f2_hbm.at[cidx[b4]], gb[b2], semg[b2]).wait()

    def _scatter_issue(b2, b4):
        pltpu.async_copy(msg[b2], accum.at[didx[b2]], sems[b2], add=True)

    def _scatter_wait(b2, b4):
        pltpu.make_async_copy(msg[b2], accum.at[didx[b2]], sems[b2]).wait()

    def _didx_fill(b2, b4):
        for v in range(CHUNK // 16):
            didx[b2][pl.ds(16 * v, 16)] = cidx[b4][pl.ds(CHUNK + 16 * v, 16)]
        if CHUNK % 16:
            didx[b2][pl.ds(CHUNK - 16, 16)] = cidx[b4][pl.ds(2 * CHUNK - 16, 16)]

    def _compute(b):
        gbb, msgb = gb[b], msg[b]

        def _edge(eg, ecarry):
            es = [eg * UNROLL + u for u in range(UNROLL)]
            rows = []
            for e in es:
                row = [gbb[e, pl.ds(16 * k, 16)] for k in range(D // 16)]
                acc0 = jnp.zeros((16,), jnp.float32)
                acc1 = jnp.zeros((16,), jnp.float32)
                for k in range(D // 16):
                    t = row[k] + gbb[CHUNK + e, pl.ds(16 * k, 16)]
                    t = jnp.maximum(t, 0.2 * t)
                    if k % 2 == 0:
                        acc0 = acc0 + t * a_sl[k]
                    else:
                        acc1 = acc1 + t * a_sl[k]
                rows.append((row, acc0 + acc1))
            for e, (row, acc) in zip(es, rows):
                logit = jnp.sum(acc)
                exv = jnp.exp(jnp.full((16,), logit, jnp.float32))
                for k in range(D // 16):
                    msgb[e, pl.ds(16 * k, 16)] = row[k] * exv
                msgb[e, pl.ds(D, 16)] = exv
            return ecarry
        lax.fori_loop(0, CHUNK // UNROLL, _edge, 0)

    def _chunk_step(i, b2, b4, has_next=True, has_next2=True,
                    wait_prev_scatter=True):
        if has_next:
            _idx_wait((b4 + 1) % 4, 1 - b2)
            _gather_issue((b4 + 1) % 4, 1 - b2)
        _gather_wait(b4, b2)
        if wait_prev_scatter:
            _scatter_wait(b2, (b4 + 2) % 4)
        _didx_fill(b2, b4)
        _compute(b2)
        _scatter_issue(b2, b4)
        if has_next2:
            _idx_issue(i + 2, (b4 + 2) % 4, b2)

    pltpu.sync_copy(ci_hbm.at[ord0], cidx0)
    _gather_issue(0, 0)
    _idx_issue(1, 1, 1)
    _chunk_step(0, 0, 0, wait_prev_scatter=False)
    _chunk_step(1, 1, 1, wait_prev_scatter=False)

    def _quad(k, carry):
        i = 2 + 4 * k
        _chunk_step(i, 0, 2)
        _chunk_step(i + 1, 1, 3)
        _chunk_step(i + 2, 0, 0)
        _chunk_step(i + 3, 1, 1)
        return carry
    lax.fori_loop(0, (NCHUNK - 6) // 4, _quad, 0)

    _chunk_step(NCHUNK - 4, 0, 2)
    _chunk_step(NCHUNK - 3, 1, 3)
    _chunk_step(NCHUNK - 2, 0, 0, has_next2=False)
    _chunk_step(NCHUNK - 1, 1, 1, has_next=False, has_next2=False)
    _scatter_wait(0, 0)
    _scatter_wait(1, 1)

    plsc.subcore_barrier()
    for t in range(15):
        pltpu.sync_copy(accum.at[pl.ds(row0 + t * CHUNK, CHUNK)],
                        out_hbm.at[cid, pl.ds(row0 + t * CHUNK, CHUNK)])
    pltpu.sync_copy(accum.at[pl.ds(row0 + 600, 25)],
                    out_hbm.at[cid, pl.ds(row0 + 600, 25)])


_RB = 2000


def _tc_matmul(x, w):
    k = w.shape[1]

    def body(x_ref, w_ref, o_ref):
        o_ref[...] = jnp.dot(x_ref[...], w_ref[...],
                             preferred_element_type=jnp.float32)

    return pl.pallas_call(
        body,
        grid=(N // _RB,),
        in_specs=[
            pl.BlockSpec((_RB, x.shape[1]), lambda i: (i, 0)),
            pl.BlockSpec((x.shape[1], k), lambda i: (0, 0)),
        ],
        out_specs=pl.BlockSpec((_RB, k), lambda i: (i, 0)),
        out_shape=jax.ShapeDtypeStruct((N, k), jnp.float32),
    )(x, w)


def _norm(acc_ref, b_ref):
    s = acc_ref[0] + acc_ref[1]
    feat = s[:, :D]
    den = s[:, D:D + 1]
    den = jnp.where(den != 0.0, den, 1.0)
    return feat / den + b_ref[...]


def _tc_norm_mm(acc, b, w):
    def body(acc_ref, b_ref, w_ref, o_ref):
        o_ref[...] = jnp.dot(_norm(acc_ref, b_ref), w_ref[...],
                             preferred_element_type=jnp.float32)

    return pl.pallas_call(
        body,
        grid=(N // _RB,),
        in_specs=[
            pl.BlockSpec((NC, _RB, W144), lambda i: (0, i, 0)),
            pl.BlockSpec((1, D), lambda i: (0, 0)),
            pl.BlockSpec((D, D), lambda i: (0, 0)),
        ],
        out_specs=pl.BlockSpec((_RB, D), lambda i: (i, 0)),
        out_shape=jax.ShapeDtypeStruct((N, D), jnp.float32),
    )(acc, b.reshape(1, D), w)


def _tc_final(acc2b, acc2l, b2b, b2l, wa, wb, outb):
    def body(a1_ref, a2_ref, b1_ref, b2_ref, wa_ref, wb_ref, ob_ref, o_ref):
        x1 = _norm(a1_ref, b1_ref)
        x2 = _norm(a2_ref, b2_ref)
        o_ref[...] = (
            jnp.dot(x1, wa_ref[...], preferred_element_type=jnp.float32)
            + jnp.dot(x2, wb_ref[...], preferred_element_type=jnp.float32)
            + ob_ref[...])

    return pl.pallas_call(
        body,
        grid=(N // _RB,),
        in_specs=[
            pl.BlockSpec((NC, _RB, W144), lambda i: (0, i, 0)),
            pl.BlockSpec((NC, _RB, W144), lambda i: (0, i, 0)),
            pl.BlockSpec((1, D), lambda i: (0, 0)),
            pl.BlockSpec((1, D), lambda i: (0, 0)),
            pl.BlockSpec((D, D), lambda i: (0, 0)),
            pl.BlockSpec((D, D), lambda i: (0, 0)),
            pl.BlockSpec((1, D), lambda i: (0, 0)),
        ],
        out_specs=pl.BlockSpec((_RB, D), lambda i: (i, 0)),
        out_shape=jax.ShapeDtypeStruct((N, D), jnp.float32),
    )(acc2b, acc2l, b2b.reshape(1, D), b2l.reshape(1, D), wa, wb,
      outb.reshape(1, D))


@jax.jit
def kernel(h_user, h_item, rate_src, rate_dst, rb_src, rb_dst, link_src,
           link_dst, W1r_src, W1r_dst, a1r, b1r, W1b_src, W1b_dst, a1b, b1b,
           W2b_src, W2b_dst, a2b, b2b, W2l_src, W2l_dst, a2l, b2l, out_W,
           out_b):
    wu = jnp.concatenate([W1r_src, W1b_dst, W2b_dst, W2l_dst], axis=1)
    wi = jnp.concatenate([W1r_dst, W1b_src], axis=1)
    mu = _tc_matmul(h_user, wu)
    mi = _tc_matmul(h_item, wi)
    fs1r, fd1b, fd2b, fd2l = (mu[:, 0:D], mu[:, D:2 * D],
                              mu[:, 2 * D:3 * D], mu[:, 3 * D:4 * D])
    fd1r, fs1b = mi[:, 0:D], mi[:, D:2 * D]

    def _ci(src, dst):
        return jnp.concatenate([src.reshape(-1, CHUNK) + N,
                                dst.reshape(-1, CHUNK)], axis=1)

    ci_rate = _ci(rate_src, rate_dst)
    ci_rb = _ci(rb_src, rb_dst)
    ci_link = _ci(link_src, link_dst)

    def _f2(fs, fd):
        return jnp.concatenate([fd, fs], axis=0)

    acc_r = _sc_edge(_f2(fs1r, fd1r), ci_rate, a1r)
    acc_b = _sc_edge(_f2(fs1b, fd1b), ci_rb, a1b)

    fs2b = _tc_norm_mm(acc_r, b1r, W2b_src)
    fs2l = _tc_norm_mm(acc_b, b1b, W2l_src)

    acc_2b = _sc_edge(_f2(fs2b, fd2b), ci_rb, a2b)
    acc_2l = _sc_edge(_f2(fs2l, fd2l), ci_link, a2l)

    return _tc_final(acc_2b, acc_2l, b2b, b2l,
                     out_W[:D, :], out_W[D:, :], out_b)


# speedup vs baseline: 20.4561x; 1.0291x over previous
import functools

import jax
import jax.numpy as jnp
from jax import lax
from jax.experimental import pallas as pl
from jax.experimental.pallas import tpu as pltpu
from jax.experimental.pallas import tpu_sc as plsc

N = 10000
D = 128
E = 320000
W144 = 144
NC = 2
NS = 16
NWORK = NC * NS
EPW = E // NWORK
CHUNK = 40
NCHUNK = EPW // CHUNK
ROWS_PER_TILE = N // NS

_mesh = plsc.VectorSubcoreMesh(
    core_axis_name="c", subcore_axis_name="s", num_cores=NC, num_subcores=NS)


UNROLL = 4


@functools.partial(
    pl.kernel,
    out_type=jax.ShapeDtypeStruct((NC, N, W144), jnp.float32),
    mesh=_mesh,
    scratch_types=(
        [pltpu.VMEM((CHUNK,), jnp.int32)] * 4
        + [pltpu.VMEM((CHUNK,), jnp.int32)] * 4
        + [pltpu.VMEM((CHUNK, D), jnp.float32)] * 2
        + [pltpu.VMEM((CHUNK, D), jnp.float32)] * 2
        + [pltpu.VMEM((CHUNK, W144), jnp.float32)] * 2
        + [pltpu.VMEM((D,), jnp.float32)]
        + [pltpu.VMEM_SHARED((N, W144), jnp.float32)]
        + [pltpu.SemaphoreType.DMA] * 6
    ),
    compiler_params=pltpu.CompilerParams(use_tc_tiling_on_sc=False, needs_layout_passes=False),
)
def _sc_edge(fs_hbm, fd_hbm, src_hbm, dst_hbm, a_hbm, out_hbm,
             sidx0, sidx1, sidx2, sidx3, didx0, didx1, didx2, didx3,
             gs0, gs1, gd0, gd1, msg0, msg1,
             avec, accum, semg0, semg1, semi0, semi1, sems0, sems1):
    cid = lax.axis_index("c")
    sid = lax.axis_index("s")
    wid = sid * NC + cid

    sidx = (sidx0, sidx1, sidx2, sidx3)
    didx = (didx0, didx1, didx2, didx3)
    gs = (gs0, gs1)
    gd = (gd0, gd1)
    msg = (msg0, msg1)
    semg = (semg0, semg1)
    semi = (semi0, semi1)
    sems = (sems0, sems1)

    pltpu.sync_copy(a_hbm, avec)

    def _zrow(i, carry):
        for j in range(W144 // 16):
            msg0[i, pl.ds(16 * j, 16)] = jnp.zeros((16,), jnp.float32)
        return carry
    lax.fori_loop(0, CHUNK, _zrow, 0)
    row0 = sid * ROWS_PER_TILE
    for t in range(15):
        pltpu.sync_copy(msg0, accum.at[pl.ds(row0 + t * CHUNK, CHUNK)])
    pltpu.sync_copy(msg0.at[pl.ds(0, 25)], accum.at[pl.ds(row0 + 600, 25)])
    plsc.subcore_barrier()

    a_sl = tuple(avec[pl.ds(16 * k, 16)] for k in range(D // 16))

    base0 = wid * EPW

    def _idx_issue(i, b4, b2):
        base = base0 + i * CHUNK
        pltpu.async_copy(src_hbm.at[pl.ds(base, CHUNK)], sidx[b4], semi[b2])
        pltpu.async_copy(dst_hbm.at[pl.ds(base, CHUNK)], didx[b4], semi[b2])

    def _idx_wait(b4, b2):
        pltpu.make_async_copy(src_hbm.at[pl.ds(0, CHUNK)], sidx[b4], semi[b2]).wait()
        pltpu.make_async_copy(dst_hbm.at[pl.ds(0, CHUNK)], didx[b4], semi[b2]).wait()

    def _gather_issue(b4, b2):
        pltpu.async_copy(fs_hbm.at[sidx[b4]], gs[b2], semg[b2])
        pltpu.async_copy(fd_hbm.at[didx[b4]], gd[b2], semg[b2])

    def _gather_wait(b4, b2):
        pltpu.make_async_copy(fs_hbm.at[sidx[b4]], gs[b2], semg[b2]).wait()
        pltpu.make_async_copy(fd_hbm.at[didx[b4]], gd[b2], semg[b2]).wait()

    def _scatter_issue(b2, b4):
        pltpu.async_copy(msg[b2], accum.at[didx[b4]], sems[b2], add=True)

    def _scatter_wait(b2, b4):
        pltpu.make_async_copy(msg[b2], accum.at[didx[b4]], sems[b2]).wait()

    def _compute(b):
        gsb, gdb, msgb = gs[b], gd[b], msg[b]

        def _edge(eg, ecarry):
            es = [eg * UNROLL + u for u in range(UNROLL)]
            rows = []
            for e in es:
                row = [gsb[e, pl.ds(16 * k, 16)] for k in range(D // 16)]
                acc0 = jnp.zeros((16,), jnp.float32)
                acc1 = jnp.zeros((16,), jnp.float32)
                for k in range(D // 16):
                    t = row[k] + gdb[e, pl.ds(16 * k, 16)]
                    t = jnp.maximum(t, 0.2 * t)
                    if k % 2 == 0:
                        acc0 = acc0 + t * a_sl[k]
                    else:
                        acc1 = acc1 + t * a_sl[k]
                rows.append((row, acc0 + acc1))
            for e, (row, acc) in zip(es, rows):
                logit = jnp.sum(acc)
                exv = jnp.exp(jnp.full((16,), logit, jnp.float32))
                for k in range(D // 16):
                    msgb[e, pl.ds(16 * k, 16)] = row[k] * exv
                msgb[e, pl.ds(D, 16)] = exv
            return ecarry
        lax.fori_loop(0, CHUNK // UNROLL, _edge, 0)

    def _chunk_step(i, b2, b4, has_next=True, has_next2=True,
                    wait_prev_scatter=True):
        if has_next:
            _idx_wait((b4 + 1) % 4, 1 - b2)
            _gather_issue((b4 + 1) % 4, 1 - b2)
        _gather_wait(b4, b2)
        if wait_prev_scatter:
            _scatter_wait(b2, (b4 + 2) % 4)
        _compute(b2)
        _scatter_issue(b2, b4)
        if has_next2:
            _idx_issue(i + 2, (b4 + 2) % 4, b2)

    pltpu.sync_copy(src_hbm.at[pl.ds(base0, CHUNK)], sidx0)
    pltpu.sync_copy(dst_hbm.at[pl.ds(base0, CHUNK)], didx0)
    _gather_issue(0, 0)
    _idx_issue(1, 1, 1)
    _chunk_step(0, 0, 0, wait_prev_scatter=False)
    _chunk_step(1, 1, 1, wait_prev_scatter=False)

    def _quad(k, carry):
        i = 2 + 4 * k
        _chunk_step(i, 0, 2)
        _chunk_step(i + 1, 1, 3)
        _chunk_step(i + 2, 0, 0)
        _chunk_step(i + 3, 1, 1)
        return carry
    lax.fori_loop(0, (NCHUNK - 6) // 4, _quad, 0)

    _chunk_step(NCHUNK - 4, 0, 2)
    _chunk_step(NCHUNK - 3, 1, 3)
    _chunk_step(NCHUNK - 2, 0, 0, has_next2=False)
    _chunk_step(NCHUNK - 1, 1, 1, has_next=False, has_next2=False)
    _scatter_wait(0, 0)
    _scatter_wait(1, 1)

    plsc.subcore_barrier()
    for t in range(15):
        pltpu.sync_copy(accum.at[pl.ds(row0 + t * CHUNK, CHUNK)],
                        out_hbm.at[cid, pl.ds(row0 + t * CHUNK, CHUNK)])
    pltpu.sync_copy(accum.at[pl.ds(row0 + 600, 25)],
                    out_hbm.at[cid, pl.ds(row0 + 600, 25)])


_RB = 2000


def _tc_matmul(x, w):
    k = w.shape[1]

    def body(x_ref, w_ref, o_ref):
        o_ref[...] = jnp.dot(x_ref[...], w_ref[...],
                             preferred_element_type=jnp.float32)

    return pl.pallas_call(
        body,
        grid=(N // _RB,),
        in_specs=[
            pl.BlockSpec((_RB, x.shape[1]), lambda i: (i, 0)),
            pl.BlockSpec((x.shape[1], k), lambda i: (0, 0)),
        ],
        out_specs=pl.BlockSpec((_RB, k), lambda i: (i, 0)),
        out_shape=jax.ShapeDtypeStruct((N, k), jnp.float32),
    )(x, w)


def _norm(acc_ref, b_ref):
    s = acc_ref[0] + acc_ref[1]
    feat = s[:, :D]
    den = s[:, D:D + 1]
    den = jnp.where(den != 0.0, den, 1.0)
    return feat / den + b_ref[...]


def _tc_norm_mm(acc, b, w):
    def body(acc_ref, b_ref, w_ref, o_ref):
        o_ref[...] = jnp.dot(_norm(acc_ref, b_ref), w_ref[...],
                             preferred_element_type=jnp.float32)

    return pl.pallas_call(
        body,
        grid=(N // _RB,),
        in_specs=[
            pl.BlockSpec((NC, _RB, W144), lambda i: (0, i, 0)),
            pl.BlockSpec((1, D), lambda i: (0, 0)),
            pl.BlockSpec((D, D), lambda i: (0, 0)),
        ],
        out_specs=pl.BlockSpec((_RB, D), lambda i: (i, 0)),
        out_shape=jax.ShapeDtypeStruct((N, D), jnp.float32),
    )(acc, b.reshape(1, D), w)


def _tc_final(acc2b, acc2l, b2b, b2l, wa, wb, outb):
    def body(a1_ref, a2_ref, b1_ref, b2_ref, wa_ref, wb_ref, ob_ref, o_ref):
        x1 = _norm(a1_ref, b1_ref)
        x2 = _norm(a2_ref, b2_ref)
        o_ref[...] = (
            jnp.dot(x1, wa_ref[...], preferred_element_type=jnp.float32)
            + jnp.dot(x2, wb_ref[...], preferred_element_type=jnp.float32)
            + ob_ref[...])

    return pl.pallas_call(
        body,
        grid=(N // _RB,),
        in_specs=[
            pl.BlockSpec((NC, _RB, W144), lambda i: (0, i, 0)),
            pl.BlockSpec((NC, _RB, W144), lambda i: (0, i, 0)),
            pl.BlockSpec((1, D), lambda i: (0, 0)),
            pl.BlockSpec((1, D), lambda i: (0, 0)),
            pl.BlockSpec((D, D), lambda i: (0, 0)),
            pl.BlockSpec((D, D), lambda i: (0, 0)),
            pl.BlockSpec((1, D), lambda i: (0, 0)),
        ],
        out_specs=pl.BlockSpec((_RB, D), lambda i: (i, 0)),
        out_shape=jax.ShapeDtypeStruct((N, D), jnp.float32),
    )(acc2b, acc2l, b2b.reshape(1, D), b2l.reshape(1, D), wa, wb,
      outb.reshape(1, D))


@jax.jit
def kernel(h_user, h_item, rate_src, rate_dst, rb_src, rb_dst, link_src,
           link_dst, W1r_src, W1r_dst, a1r, b1r, W1b_src, W1b_dst, a1b, b1b,
           W2b_src, W2b_dst, a2b, b2b, W2l_src, W2l_dst, a2l, b2l, out_W,
           out_b):
    wu = jnp.concatenate([W1r_src, W1b_dst, W2b_dst, W2l_dst], axis=1)
    wi = jnp.concatenate([W1r_dst, W1b_src], axis=1)
    mu = _tc_matmul(h_user, wu)
    mi = _tc_matmul(h_item, wi)
    fs1r, fd1b, fd2b, fd2l = (mu[:, 0:D], mu[:, D:2 * D],
                              mu[:, 2 * D:3 * D], mu[:, 3 * D:4 * D])
    fd1r, fs1b = mi[:, 0:D], mi[:, D:2 * D]

    acc_r = _sc_edge(fs1r, fd1r, rate_src, rate_dst, a1r)
    acc_b = _sc_edge(fs1b, fd1b, rb_src, rb_dst, a1b)

    fs2b = _tc_norm_mm(acc_r, b1r, W2b_src)
    fs2l = _tc_norm_mm(acc_b, b1b, W2l_src)

    acc_2b = _sc_edge(fs2b, fd2b, rb_src, rb_dst, a2b)
    acc_2l = _sc_edge(fs2l, fd2l, link_src, link_dst, a2l)

    return _tc_final(acc_2b, acc_2l, b2b, b2l,
                     out_W[:D, :], out_W[D:, :], out_b)
